# baseline (speedup 1.0000x reference)
import functools

import jax
import jax.numpy as jnp
from jax import lax
from jax.experimental import pallas as pl
from jax.experimental.pallas import tpu as pltpu
from jax.experimental.pallas import tpu_sc as plsc

LANES = 16
BL = 32
NS = 16
NC = 2
K = 96
NBUF = 3
SBUF = 2
IBUF = 6


def _perm(f):
    p = []
    for jj in range(f // BL):
        p.extend(BL * jj + 2 * k for k in range(LANES))
        p.extend(BL * jj + 2 * k + 1 for k in range(LANES))
    return p


def _mm_body(x_ref, w_ref, o_ref):
    o_ref[...] = lax.dot_general(
        x_ref[...], w_ref[...], (((1,), (1,)), ((), ())),
        preferred_element_type=jnp.float32).astype(jnp.bfloat16)


def _matmul(x, w):
    n, f = x.shape
    o = w.shape[0]
    bn = 400
    grid = n // bn
    return pl.pallas_call(
        _mm_body,
        grid=(grid,),
        in_specs=[
            pl.BlockSpec((bn, f), lambda i: (i, 0)),
            pl.BlockSpec((o, f), lambda i: (0, 0)),
        ],
        out_specs=pl.BlockSpec((bn, o), lambda i: (i, 0)),
        out_shape=jax.ShapeDtypeStruct((n, o), jnp.bfloat16),
    )(x, w)


def _sc_scatter_fn(n, t_steps, f, chunks):
    rpt = n // NS
    tp = t_steps // NC
    mask = jnp.int32(-65536)

    def body(y_hbm, pk_hbm, w_hbm, bias_hbm, out_hbm,
             acc_sh, pk_v, wv_v, rows_v, sc_v,
             gs0, gs1, gs2, ss0, ss1,
             is0, is1, is2, is3, is4, is5):
        gsems = (gs0, gs1, gs2)
        ssems = (ss0, ss1)
        isems = (is0, is1, is2, is3, is4, is5)
        c = lax.axis_index("c")
        s = lax.axis_index("s")
        base = s * rpt

        def start_idx(j6, g, tt):
            pltpu.async_copy(pk_hbm.at[s, g], pk_v.at[j6], isems[j6])
            pltpu.async_copy(w_hbm.at[tt, s, g], wv_v.at[j6], isems[j6])

        def wait_idx(j6):
            pltpu.make_async_copy(pk_hbm.at[0, 0], pk_v.at[j6], isems[j6]).wait()
            pltpu.make_async_copy(w_hbm.at[0, 0, 0], wv_v.at[j6],
                                  isems[j6]).wait()

        def start_gather(j, j6):
            pltpu.async_copy(y_hbm.at[pk_v.at[j6, 0]], rows_v.at[j], gsems[j])

        def wait_gather(j):
            pltpu.make_async_copy(y_hbm.at[pl.ds(0, K)], rows_v.at[j],
                                  gsems[j]).wait()

        def start_scatter(j2, j6):
            pltpu.async_copy(sc_v.at[j2], acc_sh.at[pk_v.at[j6, 1]],
                             ssems[j2], add=True)

        def wait_scatter(j2):
            pltpu.make_async_copy(sc_v.at[j2], acc_sh.at[pl.ds(0, K)],
                                  ssems[j2]).wait()

        def scale(j, j2, j6):
            def scale_grp(i16, c2):
                wvec = wv_v[j6, pl.ds(i16 * LANES, LANES)]
                for lane in range(LANES):
                    w = wvec[lane]
                    row = i16 * LANES + lane
                    for jj in range(f // BL):
                        vi = rows_v[j, row, pl.ds(jj * LANES, LANES)]
                        lo = lax.bitcast_convert_type(
                            lax.shift_left(vi, 16), jnp.float32)
                        hi = lax.bitcast_convert_type(
                            lax.bitwise_and(vi, mask), jnp.float32)
                        sc_v[j2, row, pl.ds(jj * BL, LANES)] = lo * w
                        sc_v[j2, row, pl.ds(jj * BL + LANES, LANES)] = hi * w
                return c2

            lax.fori_loop(0, K // LANES, scale_grp, 0)

        for p in range(tp):
            t = c * tp + p
            pltpu.sync_copy(bias_hbm, acc_sh.at[pl.ds(base, rpt)])
            plsc.subcore_barrier()

            if p == 0:
                for g0 in range(4):
                    start_idx(g0, g0, t)
            wait_idx(0)
            start_gather(0, 0)
            wait_idx(1)
            start_gather(1, 1)

            def super_body(sg, carry):
                for u in range(IBUF):
                    g = sg * IBUF + u
                    j = u % NBUF
                    j2 = u % SBUF
                    j6 = u
                    wait_gather(j)

                    @pl.when(g >= SBUF)
                    def _():
                        wait_scatter(j2)

                    scale(j, j2, j6)
                    start_scatter(j2, j6)
                    g2 = g + 2
                    jg2 = (u + 2) % NBUF
                    j62 = (u + 2) % IBUF

                    @pl.when(g2 < chunks)
                    def _():
                        wait_idx(j62)
                        start_gather(jg2, j62)

                    g4 = g + 4
                    j64 = (u + 4) % IBUF

                    @pl.when(g4 < chunks)
                    def _():
                        start_idx(j64, g4, t)
                return carry

            lax.fori_loop(0, chunks // IBUF, super_body, 0)
            for j2 in range(SBUF):
                wait_scatter(j2)
            if p + 1 < tp:
                for g0 in range(4):
                    start_idx(g0, g0, t + 1)
            plsc.subcore_barrier()
            pltpu.sync_copy(acc_sh.at[pl.ds(base, rpt)],
                            out_hbm.at[pl.ds(base, rpt), t])
        return None

    mesh = plsc.VectorSubcoreMesh(core_axis_name="c", subcore_axis_name="s")
    return pl.kernel(
        body,
        mesh=mesh,
        compiler_params=pltpu.CompilerParams(use_tc_tiling_on_sc=False),
        out_type=jax.ShapeDtypeStruct((n, t_steps, f), jnp.float32),
        scratch_types=[
            pltpu.VMEM_SHARED((n, f), jnp.float32),
            pltpu.VMEM((IBUF, 2, K), jnp.int32),
            pltpu.VMEM((IBUF, K), jnp.float32),
            pltpu.VMEM((NBUF, K, f // 2), jnp.int32),
            pltpu.VMEM((SBUF, K, f), jnp.float32),
            pltpu.SemaphoreType.DMA,
            pltpu.SemaphoreType.DMA,
            pltpu.SemaphoreType.DMA,
            pltpu.SemaphoreType.DMA,
            pltpu.SemaphoreType.DMA,
            pltpu.SemaphoreType.DMA,
            pltpu.SemaphoreType.DMA,
            pltpu.SemaphoreType.DMA,
            pltpu.SemaphoreType.DMA,
            pltpu.SemaphoreType.DMA,
            pltpu.SemaphoreType.DMA,
        ],
    )


def kernel(node_features, edge_weights, W, b, edge_index):
    n, f = node_features.shape
    t_steps, e = edge_weights.shape
    src = edge_index[0].astype(jnp.int32)
    dst = edge_index[1].astype(jnp.int32)
    chunks = -(-e // (NS * K))
    chunks = -(-chunks // IBUF) * IBUF
    e_pad = chunks * NS * K
    pad = e_pad - e
    src_p = jnp.pad(src, (0, pad)).reshape(NS, chunks, 1, K)
    dst_p = jnp.pad(dst, (0, pad)).reshape(NS, chunks, 1, K)
    pk = jnp.concatenate([src_p, dst_p], axis=2)
    w_p = jnp.pad(edge_weights, ((0, 0), (0, pad))).reshape(t_steps, NS, chunks, K)
    perm = jnp.array(_perm(f), dtype=jnp.int32)
    inv = jnp.argsort(perm)
    bias_blk = jnp.broadcast_to(b[perm], (n // NS, f))
    y = _matmul(node_features, W)
    y_pk = lax.bitcast_convert_type(y.reshape(n, f // 2, 2), jnp.int32)
    out = _sc_scatter_fn(n, t_steps, f, chunks)(y_pk, pk, w_p, bias_blk)
    return jnp.take(out, inv, axis=-1)


# speedup vs baseline: 1.8920x; 1.8920x over previous
import functools

import jax
import jax.numpy as jnp
from jax import lax
from jax.experimental import pallas as pl
from jax.experimental.pallas import tpu as pltpu
from jax.experimental.pallas import tpu_sc as plsc

LANES = 16
NS = 16
NC = 2
K = 112
NBUF = 3
IBUF = 6


def _mm_body(x_ref, w_ref, o_ref):
    o_ref[...] = lax.dot_general(
        x_ref[...], w_ref[...], (((1,), (1,)), ((), ())),
        preferred_element_type=jnp.float32)


def _matmul(x, w):
    n, f = x.shape
    o = w.shape[0]
    bn = 400
    grid = n // bn
    return pl.pallas_call(
        _mm_body,
        grid=(grid,),
        in_specs=[
            pl.BlockSpec((bn, f), lambda i: (i, 0)),
            pl.BlockSpec((o, f), lambda i: (0, 0)),
        ],
        out_specs=pl.BlockSpec((bn, o), lambda i: (i, 0)),
        out_shape=jax.ShapeDtypeStruct((n, o), jnp.float32),
    )(x, w)


def _sc_scatter_fn(n, t_steps, f, chunks):
    rpt = n // NS
    tp = t_steps // NC

    def body(y_hbm, pk_hbm, w_hbm, bias_hbm, out_hbm,
             acc_sh, pk_v, wv_v, rows_v,
             gs0, gs1, gs2, ss0, ss1, ss2,
             is0, is1, is2, is3, is4, is5):
        gsems = (gs0, gs1, gs2)
        ssems = (ss0, ss1, ss2)
        isems = (is0, is1, is2, is3, is4, is5)
        c = lax.axis_index("c")
        s = lax.axis_index("s")
        base = s * rpt

        def start_idx(j6, g, tt):
            pltpu.async_copy(pk_hbm.at[s, g], pk_v.at[j6], isems[j6])
            pltpu.async_copy(w_hbm.at[tt, s, g], wv_v.at[j6], isems[j6])

        def wait_idx(j6):
            pltpu.make_async_copy(pk_hbm.at[0, 0], pk_v.at[j6], isems[j6]).wait()
            pltpu.make_async_copy(w_hbm.at[0, 0, 0], wv_v.at[j6], isems[j6]).wait()

        def start_gather(j, j6):
            pltpu.async_copy(y_hbm.at[pk_v.at[j6, 0]], rows_v.at[j], gsems[j])

        def wait_gather(j):
            pltpu.make_async_copy(y_hbm.at[pl.ds(0, K)], rows_v.at[j],
                                  gsems[j]).wait()

        def start_scatter(j, j6):
            pltpu.async_copy(rows_v.at[j], acc_sh.at[pk_v.at[j6, 1]], ssems[j],
                             add=True)

        def wait_scatter(j):
            pltpu.make_async_copy(rows_v.at[j], acc_sh.at[pl.ds(0, K)],
                                  ssems[j]).wait()

        def scale(j, j6):
            def scale_grp(i16, c2):
                wvec = wv_v[j6, pl.ds(i16 * LANES, LANES)]
                for lane in range(LANES):
                    w = wvec[lane]
                    row = i16 * LANES + lane
                    for jj in range(f // LANES):
                        sl = pl.ds(jj * LANES, LANES)
                        rows_v[j, row, sl] = rows_v[j, row, sl] * w
                return c2

            lax.fori_loop(0, K // LANES, scale_grp, 0)

        for p in range(tp):
            t = c * tp + p
            pltpu.sync_copy(bias_hbm, acc_sh.at[pl.ds(base, rpt)])
            plsc.subcore_barrier()

            if p == 0:
                for g0 in range(4):
                    start_idx(g0, g0, t)
            wait_idx(0)
            start_gather(0, 0)
            wait_idx(1)
            start_gather(1, 1)

            def super_body(sg, carry):
                for u in range(IBUF):
                    g = sg * IBUF + u
                    j = u % NBUF
                    j6 = u
                    wait_gather(j)
                    scale(j, j6)
                    start_scatter(j, j6)
                    g2 = g + 2
                    j2 = (j + 2) % NBUF
                    j62 = (u + 2) % IBUF

                    @pl.when(g2 < chunks)
                    def _():
                        wait_idx(j62)

                        @pl.when(g2 >= NBUF)
                        def _():
                            wait_scatter(j2)

                        start_gather(j2, j62)

                    g4 = g + 4
                    j64 = (u + 4) % IBUF

                    @pl.when(g4 < chunks)
                    def _():
                        start_idx(j64, g4, t)
                return carry

            lax.fori_loop(0, chunks // IBUF, super_body, 0)
            for j in range(NBUF):
                wait_scatter(j)
            if p + 1 < tp:
                for g0 in range(4):
                    start_idx(g0, g0, t + 1)
            plsc.subcore_barrier()
            pltpu.sync_copy(acc_sh.at[pl.ds(base, rpt)],
                            out_hbm.at[pl.ds(base, rpt), t])

        return None

    mesh = plsc.VectorSubcoreMesh(core_axis_name="c", subcore_axis_name="s")
    return pl.kernel(
        body,
        mesh=mesh,
        out_type=jax.ShapeDtypeStruct((n, t_steps, f), jnp.float32),
        scratch_types=[
            pltpu.VMEM_SHARED((n, f), jnp.float32),
            pltpu.VMEM((IBUF, 2, K), jnp.int32),
            pltpu.VMEM((IBUF, K), jnp.float32),
            pltpu.VMEM((NBUF, K, f), jnp.float32),
            pltpu.SemaphoreType.DMA,
            pltpu.SemaphoreType.DMA,
            pltpu.SemaphoreType.DMA,
            pltpu.SemaphoreType.DMA,
            pltpu.SemaphoreType.DMA,
            pltpu.SemaphoreType.DMA,
            pltpu.SemaphoreType.DMA,
            pltpu.SemaphoreType.DMA,
            pltpu.SemaphoreType.DMA,
            pltpu.SemaphoreType.DMA,
            pltpu.SemaphoreType.DMA,
            pltpu.SemaphoreType.DMA,
        ],
    )


def kernel(node_features, edge_weights, W, b, edge_index):
    n, f = node_features.shape
    t_steps, e = edge_weights.shape
    src = edge_index[0].astype(jnp.int32)
    dst = edge_index[1].astype(jnp.int32)
    chunks = -(-e // (NS * K))
    chunks = -(-chunks // IBUF) * IBUF
    e_pad = chunks * NS * K
    pad = e_pad - e
    src_p = jnp.pad(src, (0, pad)).reshape(NS, chunks, 1, K)
    dst_p = jnp.pad(dst, (0, pad)).reshape(NS, chunks, 1, K)
    pk = jnp.concatenate([src_p, dst_p], axis=2)
    w_p = jnp.pad(edge_weights, ((0, 0), (0, pad))).reshape(t_steps, NS, chunks, K)
    bias_blk = jnp.broadcast_to(b, (n // NS, f))
    y = _matmul(node_features, W)
    out = _sc_scatter_fn(n, t_steps, f, chunks)(y, pk, w_p, bias_blk)
    return out


# speedup vs baseline: 1.8937x; 1.0009x over previous
import functools

import jax
import jax.numpy as jnp
from jax import lax
from jax.experimental import pallas as pl
from jax.experimental.pallas import tpu as pltpu
from jax.experimental.pallas import tpu_sc as plsc

LANES = 16
NS = 16
NC = 2
K = 112
NBUF = 3
IBUF = 6


def _mm_body(x_ref, w_ref, o_ref):
    o_ref[...] = lax.dot_general(
        x_ref[...], w_ref[...], (((1,), (1,)), ((), ())),
        preferred_element_type=jnp.float32)


def _matmul(x, w):
    n, f = x.shape
    o = w.shape[0]
    bn = 400
    grid = n // bn
    return pl.pallas_call(
        _mm_body,
        grid=(grid,),
        in_specs=[
            pl.BlockSpec((bn, f), lambda i: (i, 0)),
            pl.BlockSpec((o, f), lambda i: (0, 0)),
        ],
        out_specs=pl.BlockSpec((bn, o), lambda i: (i, 0)),
        out_shape=jax.ShapeDtypeStruct((n, o), jnp.float32),
    )(x, w)


def _sc_scatter_fn(n, t_steps, f, chunks):
    rpt = n // NS
    tp = t_steps // NC

    def body(y_hbm, pk_hbm, w_hbm, bias_hbm, out_hbm,
             acc_sh, pk_v, wv_v, rows_v,
             gs0, gs1, gs2, ss0, ss1, ss2,
             is0, is1, is2, is3, is4, is5):
        gsems = (gs0, gs1, gs2)
        ssems = (ss0, ss1, ss2)
        isems = (is0, is1, is2, is3, is4, is5)
        c = lax.axis_index("c")
        s = lax.axis_index("s")
        base = s * rpt

        def start_idx(j6, g, tt):
            pltpu.async_copy(pk_hbm.at[s, g], pk_v.at[j6], isems[j6])
            pltpu.async_copy(w_hbm.at[tt, s, g], wv_v.at[j6], isems[j6])

        def wait_idx(j6):
            pltpu.make_async_copy(pk_hbm.at[0, 0], pk_v.at[j6], isems[j6]).wait()
            pltpu.make_async_copy(w_hbm.at[0, 0, 0], wv_v.at[j6], isems[j6]).wait()

        def start_gather(j, j6):
            pltpu.async_copy(y_hbm.at[pk_v.at[j6, 0]], rows_v.at[j], gsems[j])

        def wait_gather(j):
            pltpu.make_async_copy(y_hbm.at[pl.ds(0, K)], rows_v.at[j],
                                  gsems[j]).wait()

        def start_scatter(j, j6):
            pltpu.async_copy(rows_v.at[j], acc_sh.at[pk_v.at[j6, 1]], ssems[j],
                             add=True)

        def wait_scatter(j):
            pltpu.make_async_copy(rows_v.at[j], acc_sh.at[pl.ds(0, K)],
                                  ssems[j]).wait()

        def scale(j, j6):
            def scale_grp(i16, c2):
                wvec = wv_v[j6, pl.ds(i16 * LANES, LANES)]
                for lane in range(LANES):
                    w = wvec[lane]
                    row = i16 * LANES + lane
                    for jj in range(f // LANES):
                        sl = pl.ds(jj * LANES, LANES)
                        rows_v[j, row, sl] = rows_v[j, row, sl] * w
                return c2

            lax.fori_loop(0, K // LANES, scale_grp, 0)

        for p in range(tp):
            t = c * tp + p
            if p == 0:
                pltpu.sync_copy(bias_hbm, acc_sh.at[pl.ds(base, rpt)])
                plsc.subcore_barrier()
                for g0 in range(4):
                    start_idx(g0, g0, t)
                wait_idx(0)
                start_gather(0, 0)
                wait_idx(1)
                start_gather(1, 1)

            def super_body(sg, carry):
                for u in range(IBUF):
                    g = sg * IBUF + u
                    j = u % NBUF
                    j6 = u
                    wait_gather(j)
                    scale(j, j6)
                    start_scatter(j, j6)
                    g2 = g + 2
                    j2 = (j + 2) % NBUF
                    j62 = (u + 2) % IBUF

                    @pl.when(g2 < chunks)
                    def _():
                        wait_idx(j62)

                        @pl.when(g2 >= NBUF)
                        def _():
                            wait_scatter(j2)

                        start_gather(j2, j62)

                    g4 = g + 4
                    j64 = (u + 4) % IBUF

                    @pl.when(g4 < chunks)
                    def _():
                        start_idx(j64, g4, t)
                return carry

            lax.fori_loop(0, chunks // IBUF, super_body, 0)
            for j in range(NBUF):
                wait_scatter(j)
            if p + 1 < tp:
                for g0 in range(4):
                    start_idx(g0, g0, t + 1)
                wait_idx(0)
                start_gather(0, 0)
                wait_idx(1)
                start_gather(1, 1)
            plsc.subcore_barrier()
            pltpu.sync_copy(acc_sh.at[pl.ds(base, rpt)],
                            out_hbm.at[pl.ds(base, rpt), t])
            if p + 1 < tp:
                pltpu.sync_copy(bias_hbm, acc_sh.at[pl.ds(base, rpt)])
                plsc.subcore_barrier()

        return None

    mesh = plsc.VectorSubcoreMesh(core_axis_name="c", subcore_axis_name="s")
    return pl.kernel(
        body,
        mesh=mesh,
        out_type=jax.ShapeDtypeStruct((n, t_steps, f), jnp.float32),
        scratch_types=[
            pltpu.VMEM_SHARED((n, f), jnp.float32),
            pltpu.VMEM((IBUF, 2, K), jnp.int32),
            pltpu.VMEM((IBUF, K), jnp.float32),
            pltpu.VMEM((NBUF, K, f), jnp.float32),
            pltpu.SemaphoreType.DMA,
            pltpu.SemaphoreType.DMA,
            pltpu.SemaphoreType.DMA,
            pltpu.SemaphoreType.DMA,
            pltpu.SemaphoreType.DMA,
            pltpu.SemaphoreType.DMA,
            pltpu.SemaphoreType.DMA,
            pltpu.SemaphoreType.DMA,
            pltpu.SemaphoreType.DMA,
            pltpu.SemaphoreType.DMA,
            pltpu.SemaphoreType.DMA,
            pltpu.SemaphoreType.DMA,
        ],
    )


def kernel(node_features, edge_weights, W, b, edge_index):
    n, f = node_features.shape
    t_steps, e = edge_weights.shape
    src = edge_index[0].astype(jnp.int32)
    dst = edge_index[1].astype(jnp.int32)
    chunks = -(-e // (NS * K))
    chunks = -(-chunks // IBUF) * IBUF
    e_pad = chunks * NS * K
    pad = e_pad - e
    src_p = jnp.pad(src, (0, pad)).reshape(NS, chunks, 1, K)
    dst_p = jnp.pad(dst, (0, pad)).reshape(NS, chunks, 1, K)
    pk = jnp.concatenate([src_p, dst_p], axis=2)
    w_p = jnp.pad(edge_weights, ((0, 0), (0, pad))).reshape(t_steps, NS, chunks, K)
    bias_blk = jnp.broadcast_to(b, (n // NS, f))
    y = _matmul(node_features, W)
    out = _sc_scatter_fn(n, t_steps, f, chunks)(y, pk, w_p, bias_blk)
    return out
